# Initial kernel scaffold; baseline (speedup 1.0000x reference)
import jax
import jax.numpy as jnp
from jax.experimental import pallas as pl


def kernel(x, emb):
    raise NotImplementedError("write your pallas kernel here")


# speedup vs baseline: 1.6721x; 1.6721x over previous
import jax
import jax.numpy as jnp
from jax.experimental import pallas as pl

_TB = 1024


def _add_block(x_ref, emb_ref, o_ref):
    o_ref[...] = x_ref[...] + emb_ref[...]


def kernel(x, emb):
    B, T, D = x.shape
    grid = (T // _TB, B)
    return pl.pallas_call(
        _add_block,
        grid=grid,
        in_specs=[
            pl.BlockSpec((1, _TB, D), lambda i, j: (j, i, 0)),
            pl.BlockSpec((_TB, D), lambda i, j: (i, 0)),
        ],
        out_specs=pl.BlockSpec((1, _TB, D), lambda i, j: (j, i, 0)),
        out_shape=jax.ShapeDtypeStruct(x.shape, x.dtype),
    )(x, emb)


# speedup vs baseline: 1.7381x; 1.0395x over previous
import jax
import jax.numpy as jnp
from jax.experimental import pallas as pl

_TB = 2048


def _add_block(x_ref, emb_ref, o_ref):
    o_ref[...] = x_ref[...] + emb_ref[...]


def kernel(x, emb):
    B, T, D = x.shape
    grid = (T // _TB, B)
    return pl.pallas_call(
        _add_block,
        grid=grid,
        in_specs=[
            pl.BlockSpec((1, _TB, D), lambda i, j: (j, i, 0)),
            pl.BlockSpec((_TB, D), lambda i, j: (i, 0)),
        ],
        out_specs=pl.BlockSpec((1, _TB, D), lambda i, j: (j, i, 0)),
        out_shape=jax.ShapeDtypeStruct(x.shape, x.dtype),
    )(x, emb)
